# baseline (speedup 1.0000x reference)
import functools

import jax
import jax.numpy as jnp
from jax import lax
from jax.experimental import pallas as pl
from jax.experimental.pallas import tpu as pltpu
from jax.experimental.pallas import tpu_sc as plsc

NSPLIT = 2


def _sc_gather_half(W, tok, row0, PR, NC, NS, L):
    P, V = W.shape
    N = tok.shape[0]
    NW = NC * NS
    assert PR == NW
    CH = 2048
    NCH = N // CH
    assert NCH % 2 == 0

    mesh = plsc.VectorSubcoreMesh(core_axis_name="c", subcore_axis_name="s")

    @functools.partial(
        pl.kernel,
        mesh=mesh,
        compiler_params=pltpu.CompilerParams(needs_layout_passes=False),
        out_type=jax.ShapeDtypeStruct((PR, N), jnp.float32),
        scratch_types=[
            pltpu.VMEM((V,), jnp.float32),
            pltpu.VMEM((N,), jnp.int32),
            pltpu.VMEM((CH,), jnp.float32),
            pltpu.VMEM((CH,), jnp.float32),
            pltpu.SemaphoreType.DMA,
            pltpu.SemaphoreType.DMA,
        ],
    )
    def sc_kernel(w_hbm, tok_hbm, ut_hbm, wrow, idxs, ub0, ub1, su0, su1):
        wid = lax.axis_index("s") * NC + lax.axis_index("c")
        pltpu.sync_copy(tok_hbm, idxs)
        bufs = ((ub0, su0), (ub1, su1))
        p = wid + row0
        pltpu.sync_copy(w_hbm.at[p], wrow)

        def pair_body(k, carry):
            for b, (ub, su) in enumerate(bufs):
                c = k * 2 + b

                @pl.when(k > 0)
                def _wait_prev():
                    pltpu.make_async_copy(
                        ub, ut_hbm.at[wid, pl.ds(0, CH)], su
                    ).wait()

                base = c * CH

                @plsc.parallel_loop(0, CH // L, 1, unroll=8)
                def _gather_body(j, base=base, ub=ub):
                    off = j * L
                    idx = idxs[pl.ds(base + off, L)]
                    ub[pl.ds(off, L)] = plsc.load_gather(wrow, [idx])
                pltpu.async_copy(ub, ut_hbm.at[wid, pl.ds(base, CH)], su)
            return carry

        lax.fori_loop(0, NCH // 2, pair_body, 0)
        for ub, su in bufs:
            pltpu.make_async_copy(ub, ut_hbm.at[wid, pl.ds(0, CH)], su).wait()

    return sc_kernel(W, tok)


def _tc_expand_half(ut_h, fvec_h, spc_h, h, P, prev, BN):
    PR, N = ut_h.shape
    KA = fvec_h.shape[1]
    A = KA // PR
    PA = P * A

    def tc_body(ut_ref, fvec_ref, spc_ref, *refs):
        if prev is not None:
            refs = refs[3:]
        ut_out_ref, ft_ref, y_ref, e_ref = refs

        @pl.when(pl.program_id(0) == 0)
        def _build_e():
            col = lax.broadcasted_iota(jnp.int32, (PR, KA), 1)
            row = lax.broadcasted_iota(jnp.int32, (PR, KA), 0)
            e_ref[...] = jnp.where(
                (col // A) == row,
                jnp.broadcast_to(fvec_ref[...], (PR, KA)),
                0.0,
            )

        ut_blk = ut_ref[...]
        fh = (ut_blk >= spc_ref[...]).astype(jnp.float32)
        tt = fh * ut_blk
        ut_out_ref[...] = ut_blk
        ft_ref[...] = fh
        y_ref[...] = jax.lax.dot_general(
            e_ref[...], tt,
            (((0,), (0,)), ((), ())),
            precision=jax.lax.Precision.DEFAULT,
            preferred_element_type=jnp.float32,
        )

    in_specs = [
        pl.BlockSpec((PR, BN), lambda i: (0, i)),
        pl.BlockSpec((1, KA), lambda i: (0, 0)),
        pl.BlockSpec((PR, 1), lambda i: (0, 0)),
    ]
    operands = [ut_h, fvec_h, spc_h]
    aliases = {}
    if prev is not None:
        in_specs += [pl.BlockSpec(memory_space=pl.ANY)] * 3
        operands += list(prev)
        aliases = {3: 0, 4: 1, 5: 2}

    return pl.pallas_call(
        tc_body,
        grid=(N // BN,),
        in_specs=in_specs,
        out_specs=[
            pl.BlockSpec((PR, BN), lambda i: (h, i)),
            pl.BlockSpec((PR, BN), lambda i: (h, i)),
            pl.BlockSpec((KA, BN), lambda i: (h, i)),
        ],
        out_shape=[
            jax.ShapeDtypeStruct((P, N), jnp.float32),
            jax.ShapeDtypeStruct((P, N), jnp.float32),
            jax.ShapeDtypeStruct((PA, N), jnp.float32),
        ],
        input_output_aliases=aliases,
        scratch_shapes=[pltpu.VMEM((PR, KA), jnp.float32)],
    )(*operands)


def kernel(input_ids, W, f, SP):
    P, V = W.shape
    A = f.shape[1]
    tok = input_ids.reshape(-1)
    N = tok.shape[0]

    info = plsc.get_sparse_core_info()
    NW = info.num_cores * info.num_subcores
    PR = P // NSPLIT
    assert PR == NW

    fvec = f.reshape(1, P * A)
    spc = SP.reshape(P, 1)

    prev = None
    for h in range(NSPLIT):
        ut_h = _sc_gather_half(
            W, tok, h * PR, PR,
            info.num_cores, info.num_subcores, info.num_lanes,
        )
        prev = _tc_expand_half(
            ut_h,
            fvec[:, h * PR * A:(h + 1) * PR * A],
            spc[h * PR:(h + 1) * PR],
            h, P, prev, BN=4096,
        )

    ut, ft, z = prev
    y = z.reshape(P, A, N).transpose(2, 0, 1)
    return ut.T, ft.T, y


# speedup vs baseline: 1.1198x; 1.1198x over previous
import functools

import jax
import jax.numpy as jnp
from jax import lax
from jax.experimental import pallas as pl
from jax.experimental.pallas import tpu as pltpu
from jax.experimental.pallas import tpu_sc as plsc


def _sc_gather(W, tok, NC, NS, L):
    P, V = W.shape
    N = tok.shape[0]
    NW = NC * NS
    ROWS = P // NW
    CH = 2048
    NCH = N // CH
    assert NCH % 2 == 0

    mesh = plsc.VectorSubcoreMesh(core_axis_name="c", subcore_axis_name="s")

    @functools.partial(
        pl.kernel,
        mesh=mesh,
        compiler_params=pltpu.CompilerParams(needs_layout_passes=False),
        out_type=jax.ShapeDtypeStruct((P, N), jnp.float32),
        scratch_types=[
            pltpu.VMEM((V,), jnp.float32),
            pltpu.VMEM((N,), jnp.int32),
            pltpu.VMEM((CH,), jnp.float32),
            pltpu.VMEM((CH,), jnp.float32),
            pltpu.SemaphoreType.DMA,
            pltpu.SemaphoreType.DMA,
        ],
    )
    def sc_kernel(w_hbm, tok_hbm, ut_hbm, wrow, idxs, ub0, ub1, su0, su1):
        wid = lax.axis_index("s") * NC + lax.axis_index("c")
        pltpu.sync_copy(tok_hbm, idxs)
        bufs = ((ub0, su0), (ub1, su1))
        for r in range(ROWS):
            p = wid * ROWS + r
            pltpu.sync_copy(w_hbm.at[p], wrow)

            def pair_body(k, carry, p=p):
                for b, (ub, su) in enumerate(bufs):
                    c = k * 2 + b

                    @pl.when(k > 0)
                    def _wait_prev():
                        pltpu.make_async_copy(
                            ub, ut_hbm.at[p, pl.ds(0, CH)], su
                        ).wait()

                    base = c * CH

                    @plsc.parallel_loop(0, CH // L, 1, unroll=8)
                    def _gather_body(j, base=base, ub=ub):
                        off = j * L
                        idx = idxs[pl.ds(base + off, L)]
                        ub[pl.ds(off, L)] = plsc.load_gather(wrow, [idx])

                    pltpu.async_copy(ub, ut_hbm.at[p, pl.ds(base, CH)], su)
                return carry

            lax.fori_loop(0, NCH // 2, pair_body, 0)
            for ub, su in bufs:
                pltpu.make_async_copy(
                    ub, ut_hbm.at[p, pl.ds(0, CH)], su
                ).wait()

    return sc_kernel(W, tok)


def _tc_expand(ut, fvec, spc, BN):
    P, N = ut.shape
    PA = fvec.shape[1]
    A = PA // P

    def tc_body(ut_ref, fvec_ref, spc_ref, ft_ref, y_ref, e_ref):
        @pl.when(pl.program_id(0) == 0)
        def _build_e():
            col = lax.broadcasted_iota(jnp.int32, (P, PA), 1)
            row = lax.broadcasted_iota(jnp.int32, (P, PA), 0)
            e_ref[...] = jnp.where(
                (col // A) == row,
                jnp.broadcast_to(fvec_ref[...], (P, PA)),
                0.0,
            )

        ut_blk = ut_ref[...]
        fh = (ut_blk >= spc_ref[...]).astype(jnp.float32)
        ft_ref[...] = fh
        y_ref[...] = jax.lax.dot_general(
            e_ref[...], fh * ut_blk,
            (((0,), (0,)), ((), ())),
            precision=jax.lax.Precision.DEFAULT,
            preferred_element_type=jnp.float32,
        )

    return pl.pallas_call(
        tc_body,
        grid=(N // BN,),
        in_specs=[
            pl.BlockSpec((P, BN), lambda i: (0, i)),
            pl.BlockSpec((1, PA), lambda i: (0, 0)),
            pl.BlockSpec((P, 1), lambda i: (0, 0)),
        ],
        out_specs=[
            pl.BlockSpec((P, BN), lambda i: (0, i)),
            pl.BlockSpec((PA, BN), lambda i: (0, i)),
        ],
        out_shape=[
            jax.ShapeDtypeStruct((P, N), jnp.float32),
            jax.ShapeDtypeStruct((PA, N), jnp.float32),
        ],
        scratch_shapes=[pltpu.VMEM((P, PA), jnp.float32)],
    )(ut, fvec, spc)


def kernel(input_ids, W, f, SP):
    P, V = W.shape
    A = f.shape[1]
    tok = input_ids.reshape(-1)
    N = tok.shape[0]

    info = plsc.get_sparse_core_info()
    ut = _sc_gather(W, tok, info.num_cores, info.num_subcores, info.num_lanes)

    fvec = f.reshape(1, P * A)
    spc = SP.reshape(P, 1)
    ft, z = _tc_expand(ut, fvec, spc, BN=2048)
    y = z.reshape(P, A, N).transpose(2, 0, 1)
    return ut.T, ft.T, y
